# Initial kernel scaffold; baseline (speedup 1.0000x reference)
import jax
import jax.numpy as jnp
from jax.experimental import pallas as pl


def kernel(rays_o, rays_d, grid, atoms, grid_id):
    raise NotImplementedError("write your pallas kernel here")


# speedup vs baseline: 10.2583x; 10.2583x over previous
import functools

import jax
import jax.numpy as jnp
from jax import lax
from jax.experimental import pallas as pl
from jax.experimental.pallas import tpu as pltpu
from jax.experimental.pallas import tpu_sc as plsc

_C0 = 0.28209479177387814
_C1 = 0.4886025119029199

COARSE = 64
FINE = 4
NUM_ATOMS = 16
DATA_DIM = 13
RADIUS = 1.3
COARSE_VLEN = RADIUS * 2.0 / COARSE
FINE_VLEN = COARSE_VLEN / FINE
STEP = FINE_VLEN / 2.0
N_INTRS = COARSE * 3 * 2 * FINE
N_RAYS = 64
P = N_RAYS * N_INTRS
GRID_MAX = COARSE * FINE - 1

_OFFS = ((-1, -1, -1), (-1, -1, 1), (-1, 1, -1), (-1, 1, 1),
         (1, -1, -1), (1, -1, 1), (1, 1, -1), (1, 1, 1))

NW = 32
RAYS_PER_W = N_RAYS // NW
PTS_PER_W = RAYS_PER_W * N_INTRS
CHUNKS_PER_W = PTS_PER_W // 16
CHUNKS_PER_RAY = N_INTRS // 16


def _sc_gather_kernel(pack_hbm, grid_hbm,
                      c_out, fi_out, w_out,
                      pack_v, rowbufs,
                      fi_buf, w_buf, gsem, ssem):
    ncores = 2
    wid = lax.axis_index("s") * ncores + lax.axis_index("c")
    wbase = wid * PTS_PER_W

    pltpu.sync_copy(pack_hbm, pack_v)

    lane_i = lax.broadcasted_iota(jnp.int32, (16,), 0)

    def chunk_body(j, carry):
        ray = wid * RAYS_PER_W + j // CHUNKS_PER_RAY
        c = j % CHUNKS_PER_RAY
        rp = pack_v[ray]
        ox = rp[0]
        oy = rp[1]
        oz = rp[2]
        dx = rp[3]
        dy = rp[4]
        dz = rp[5]
        ix = rp[6]
        iy = rp[7]
        iz = rp[8]
        sx = jnp.minimum((RADIUS - ox) * ix, (-RADIUS - ox) * ix)
        sy = jnp.minimum((RADIUS - oy) * iy, (-RADIUS - oy) * iy)
        sz = jnp.minimum((RADIUS - oz) * iz, (-RADIUS - oz) * iz)
        start = jnp.maximum(jnp.maximum(sx, sy), sz)

        s_idx = c * 16 + lane_i
        tvec = start + s_idx.astype(jnp.float32) * STEP
        px = ox + tvec * dx
        py = oy + tvec * dy
        pz = oz + tvec * dz
        inb = ((px > -RADIUS) & (px < RADIUS) &
               (py > -RADIUS) & (py < RADIUS) &
               (pz > -RADIUS) & (pz < RADIUS) &
               (s_idx < (N_INTRS - 1)))
        wm = jnp.where(inb, 1.0, 0.0).astype(jnp.float32)
        inv = 1.0 / FINE_VLEN
        xs = (px + RADIUS) * inv
        ys = (py + RADIUS) * inv
        zs = (pz + RADIUS) * inv

        descs = []
        for n in range(8):
            w = wm
            pfis = []
            for coord, o3 in ((xs, _OFFS[n][0]), (ys, _OFFS[n][1]),
                              (zs, _OFFS[n][2])):
                pre = coord + (o3 * 0.5)
                pf = jnp.minimum(jnp.maximum(pre, 0.0), float(GRID_MAX))
                pfi = pf.astype(jnp.int32)
                pfi = jnp.minimum(jnp.maximum(pfi, 0), GRID_MAX)
                pff = pfi.astype(jnp.float32)
                w = w * (1.0 - jnp.abs(coord - (pff + 0.5)))
                pfis.append(pfi)
            cx = pfis[0] >> 2
            cy = pfis[1] >> 2
            cz = pfis[2] >> 2
            ci = (cx * COARSE + cy) * COARSE + cz
            fi = ((pfis[0] & 3) * FINE + (pfis[1] & 3)) * FINE + (pfis[2] & 3)
            fi_buf[n, pl.ds(j * 16, 16)] = fi
            w_buf[n, pl.ds(j * 16, 16)] = w
            descs.append(pltpu.async_copy(grid_hbm.at[ci], rowbufs.at[n], gsem))
        for d in descs:
            d.wait()

        pbase = pl.multiple_of(wbase + j * 16, 16)
        out_descs = [
            pltpu.async_copy(rowbufs.at[n],
                             c_out.at[n, pl.ds(pbase, 16), :], ssem)
            for n in range(8)
        ]
        for d in out_descs:
            d.wait()
        return carry

    lax.fori_loop(0, CHUNKS_PER_W, chunk_body, 0)

    wb = pl.multiple_of(wbase, 128)
    pltpu.sync_copy(fi_buf, fi_out.at[:, pl.ds(wb, PTS_PER_W)])
    pltpu.sync_copy(w_buf, w_out.at[:, pl.ds(wb, PTS_PER_W)])


def _sc_gather(rays_o, rays_d, grid):
    pack = jnp.concatenate(
        [rays_o, rays_d, 1.0 / rays_d, jnp.zeros((N_RAYS, 7), jnp.float32)],
        axis=1)
    mesh = plsc.VectorSubcoreMesh(core_axis_name="c", subcore_axis_name="s")
    f = functools.partial(
        pl.kernel,
        mesh=mesh,
        compiler_params=pltpu.CompilerParams(use_tc_tiling_on_sc=False),
        out_type=[
            jax.ShapeDtypeStruct((8, P, NUM_ATOMS), jnp.float32),
            jax.ShapeDtypeStruct((8, P), jnp.int32),
            jax.ShapeDtypeStruct((8, P), jnp.float32),
        ],
        scratch_types=[
            pltpu.VMEM((N_RAYS, 16), jnp.float32),
            pltpu.VMEM((8, 16, 16), jnp.float32),
            pltpu.VMEM((8, PTS_PER_W), jnp.int32),
            pltpu.VMEM((8, PTS_PER_W), jnp.float32),
            pltpu.SemaphoreType.DMA,
            pltpu.SemaphoreType.DMA,
        ],
    )(_sc_gather_kernel)
    return f(pack, grid)


BP = 4096


def _tc_contract_kernel(c_ref, fi_ref, w_ref, aft_ref, out_ref):
    aft = aft_ref[...]
    eye16 = (lax.broadcasted_iota(jnp.int32, (16, 16), 0)
             == lax.broadcasted_iota(jnp.int32, (16, 16), 1)).astype(jnp.float32)
    acc = jnp.zeros((DATA_DIM, BP), jnp.float32)
    iota64 = lax.broadcasted_iota(jnp.int32, (64, BP), 0)
    for n in range(8):
        fi_n = fi_ref[n]
        oh = (iota64 == fi_n[None, :]).astype(jnp.float32)
        aselT = jax.lax.dot(aft, oh, precision=lax.Precision.HIGHEST)
        cn = c_ref[n]
        cnt = lax.dot_general(eye16, cn, (((1,), (1,)), ((), ())),
                              precision=lax.Precision.HIGHEST)
        con = jnp.zeros((DATA_DIM, BP), jnp.float32)
        for a in range(NUM_ATOMS):
            con = con + aselT[a * DATA_DIM:(a + 1) * DATA_DIM, :] * cnt[a:a + 1, :]
        acc = acc + w_ref[n][None, :] * con
    out_ref[...] = acc


def _tc_contract(c_all, fi_all, w_all, atoms_t):
    nblk = P // BP
    return pl.pallas_call(
        _tc_contract_kernel,
        grid=(nblk,),
        in_specs=[
            pl.BlockSpec((8, BP, NUM_ATOMS), lambda i: (0, i, 0)),
            pl.BlockSpec((8, BP), lambda i: (0, i)),
            pl.BlockSpec((8, BP), lambda i: (0, i)),
            pl.BlockSpec((NUM_ATOMS * DATA_DIM, 64), lambda i: (0, 0)),
        ],
        out_specs=pl.BlockSpec((DATA_DIM, BP), lambda i: (0, i)),
        out_shape=jax.ShapeDtypeStruct((DATA_DIM, P), jnp.float32),
    )(c_all, fi_all, w_all, atoms_t)


def _tc_raymarch_kernel(di_ref, rd_ref, out_ref):
    rd = rd_ref[...]
    dn = jnp.sqrt(jnp.sum(rd * rd, axis=1, keepdims=True))
    sigma = jnp.maximum(di_ref[DATA_DIM - 1], 0.0)
    alpha = 1.0 - jnp.exp(-sigma * (STEP * dn))
    logv = jnp.log(1.0 - alpha + 1e-10)
    r_i = lax.broadcasted_iota(jnp.int32, (N_INTRS, N_INTRS), 0)
    c_i = lax.broadcasted_iota(jnp.int32, (N_INTRS, N_INTRS), 1)
    m_strict = (r_i < c_i).astype(jnp.float32)
    trans = jnp.exp(jax.lax.dot(logv, m_strict,
                                precision=lax.Precision.HIGHEST))
    absl = alpha * trans
    sh = (jnp.full((N_RAYS, 1), _C0, jnp.float32),
          -_C1 * rd[:, 1:2], _C1 * rd[:, 2:3], -_C1 * rd[:, 0:1])
    cols = []
    for ch in range(3):
        pre = (sh[0] * di_ref[ch * 4 + 0] + sh[1] * di_ref[ch * 4 + 1] +
               sh[2] * di_ref[ch * 4 + 2] + sh[3] * di_ref[ch * 4 + 3])
        rgb = 1.0 / (1.0 + jnp.exp(-pre))
        cols.append(jnp.sum(absl * rgb, axis=1, keepdims=True))
    out_ref[...] = jnp.concatenate(cols, axis=1)


def _tc_raymarch(di3, rays_d):
    return pl.pallas_call(
        _tc_raymarch_kernel,
        grid=(1,),
        in_specs=[
            pl.BlockSpec((DATA_DIM, N_RAYS, N_INTRS), lambda i: (0, 0, 0)),
            pl.BlockSpec((N_RAYS, 3), lambda i: (0, 0)),
        ],
        out_specs=pl.BlockSpec((N_RAYS, 3), lambda i: (0, 0)),
        out_shape=jax.ShapeDtypeStruct((N_RAYS, 3), jnp.float32),
    )(di3, rays_d)


def kernel(rays_o, rays_d, grid, atoms, grid_id):
    del grid_id
    atoms_t = atoms.transpose(1, 2, 0).reshape(NUM_ATOMS * DATA_DIM, FINE ** 3)
    c_all, fi_all, w_all = _sc_gather(rays_o, rays_d, grid)
    di = _tc_contract(c_all, fi_all, w_all, atoms_t)
    di3 = di.reshape(DATA_DIM, N_RAYS, N_INTRS)
    return _tc_raymarch(di3, rays_d)
